# baseline (speedup 1.0000x reference)
import functools

import jax
import jax.numpy as jnp
import numpy as np
from jax.experimental import pallas as pl
from jax.experimental.pallas import tpu as pltpu


def _fold_idx_conv(O, C, H, W):
    Ho, Wo = H - 1, W - 1
    k = np.arange(C * H * W)
    c, u, v = k // (H * W), (k // W) % H, k % W
    m = np.arange(O * Ho * Wo)
    o, i_, j_ = m // (Ho * Wo), (m // Wo) % Ho, m % Wo
    di = u[:, None] - i_[None, :]
    dj = v[:, None] - j_[None, :]
    mask = (di >= 0) & (di <= 1) & (dj >= 0) & (dj <= 1)
    idx = ((o[None, :] * C + c[:, None]) * 2 + np.clip(di, 0, 1)) * 2 \
        + np.clip(dj, 0, 1)
    return idx.astype(np.int32), mask.astype(np.float32)


def _fold_idx_conv1_pooled(O=16, C=3):
    k = np.arange(C * 64)
    c, u, v = k // 64, (k // 8) % 8, k % 8
    idx = np.zeros((C * 64, 1024), dtype=np.int32)
    mask = np.zeros((C * 64, 1024), dtype=np.float32)
    for wi, (dy, dx) in enumerate([(0, 0), (0, 1), (1, 0), (1, 1)]):
        m = np.arange(O * 9)
        o, pi, pj = m // 9, (m // 3) % 3, m % 3
        i_ = 2 * pi + dy
        j_ = 2 * pj + dx
        di = u[:, None] - i_[None, :]
        dj = v[:, None] - j_[None, :]
        ok = (di >= 0) & (di <= 1) & (dj >= 0) & (dj <= 1)
        ix = ((o[None, :] * C + c[:, None]) * 2 + np.clip(di, 0, 1)) * 2 \
            + np.clip(dj, 0, 1)
        idx[:, 256 * wi:256 * wi + O * 9] = ix
        mask[:, 256 * wi:256 * wi + O * 9] = ok
    return idx, mask


_IDX1, _MASK1 = _fold_idx_conv1_pooled()
_IDX2, _MASK2 = _fold_idx_conv(32, 16, 3, 3)
_IDX3, _MASK3 = _fold_idx_conv(64, 32, 2, 2)


def _enc_kernel(x_hbm, w0_ref, b0_ref, a1_ref, c1_ref, a2_ref, c2_ref,
                a3_ref, b3_ref, o_ref, xbuf, sem, *, block_b):
    i = pl.program_id(0)
    nb = pl.num_programs(0)
    slot = jax.lax.rem(i, 2)
    nxt = jax.lax.rem(i + 1, 2)

    @pl.when(i == 0)
    def _():
        pltpu.make_async_copy(
            x_hbm.at[pl.ds(0, block_b), :], xbuf.at[0], sem.at[0]).start()

    @pl.when(i + 1 < nb)
    def _():
        pltpu.make_async_copy(
            x_hbm.at[pl.ds((i + 1) * block_b, block_b), :], xbuf.at[nxt],
            sem.at[nxt]).start()

    pltpu.make_async_copy(
        x_hbm.at[pl.ds(i * block_b, block_b), :], xbuf.at[slot],
        sem.at[slot]).wait()

    x = xbuf[slot].astype(jnp.bfloat16)
    h0 = jnp.dot(x, w0_ref[...], preferred_element_type=jnp.float32) \
        + b0_ref[...]
    t = jnp.dot(h0.astype(jnp.bfloat16), a1_ref[...],
                preferred_element_type=jnp.float32)
    m = jnp.maximum(jnp.maximum(t[:, 0:144], t[:, 256:400]),
                    jnp.maximum(t[:, 512:656], t[:, 768:912]))
    p = jnp.maximum(m + c1_ref[...], 0.0)
    h2 = jnp.maximum(
        jnp.dot(p.astype(jnp.bfloat16), a2_ref[...],
                preferred_element_type=jnp.float32) + c2_ref[...], 0.0)
    o_ref[...] = jnp.maximum(
        jnp.dot(h2.astype(jnp.bfloat16), a3_ref[...],
                preferred_element_type=jnp.float32) + b3_ref[...], 0.0)


@functools.partial(jax.jit, static_argnames=("block_b", "interpret"))
def _encode(x, W0, b0, w1, b1, w2, b2, w3, b3, block_b=2048,
            interpret=False):
    B, D = x.shape
    W0t = W0.T.astype(jnp.bfloat16)
    a1 = (w1.ravel()[_IDX1] * _MASK1).astype(jnp.bfloat16)
    a2 = (w2.ravel()[_IDX2] * _MASK2).astype(jnp.bfloat16)
    a3 = (w3.ravel()[_IDX3] * _MASK3).astype(jnp.bfloat16)
    c1 = jnp.repeat(b1, 9).reshape(1, 144)
    c2 = jnp.repeat(b2, 4).reshape(1, 128)

    nb = B // block_b
    full = lambda *s: pl.BlockSpec(s, lambda i: (0,) * len(s))
    out = pl.pallas_call(
        functools.partial(_enc_kernel, block_b=block_b),
        grid=(nb,),
        in_specs=[
            pl.BlockSpec(memory_space=pltpu.MemorySpace.HBM),
            full(D, 192),
            full(1, 192),
            full(192, 1024),
            full(1, 144),
            full(144, 128),
            full(1, 128),
            full(128, 64),
            full(1, 64),
        ],
        out_specs=pl.BlockSpec((block_b, 64), lambda i: (i, 0)),
        out_shape=jax.ShapeDtypeStruct((B, 64), jnp.float32),
        scratch_shapes=[
            pltpu.VMEM((2, block_b, D), x.dtype),
            pltpu.SemaphoreType.DMA((2,)),
        ],
        compiler_params=pltpu.CompilerParams(
            dimension_semantics=("arbitrary",)),
        interpret=interpret,
    )(x, W0t, b0.reshape(1, -1), a1, c1, a2, c2, a3, b3.reshape(1, -1))
    return out.reshape(B, 64, 1, 1)


def kernel(x, W0, b0, w1, b1, w2, b2, w3, b3):
    return _encode(x, W0, b0, w1, b1, w2, b2, w3, b3)


# speedup vs baseline: 1.0131x; 1.0131x over previous
import functools

import jax
import jax.numpy as jnp
import numpy as np
from jax.experimental import pallas as pl
from jax.experimental.pallas import tpu as pltpu


def _fold_idx_conv(O, C, H, W):
    Ho, Wo = H - 1, W - 1
    k = np.arange(C * H * W)
    c, u, v = k // (H * W), (k // W) % H, k % W
    m = np.arange(O * Ho * Wo)
    o, i_, j_ = m // (Ho * Wo), (m // Wo) % Ho, m % Wo
    di = u[:, None] - i_[None, :]
    dj = v[:, None] - j_[None, :]
    mask = (di >= 0) & (di <= 1) & (dj >= 0) & (dj <= 1)
    idx = ((o[None, :] * C + c[:, None]) * 2 + np.clip(di, 0, 1)) * 2 \
        + np.clip(dj, 0, 1)
    return idx.astype(np.int32), mask.astype(np.float32)


def _fold_idx_conv1_pooled(O=16, C=3):
    k = np.arange(C * 64)
    c, u, v = k // 64, (k // 8) % 8, k % 8
    idx = np.zeros((C * 64, 1024), dtype=np.int32)
    mask = np.zeros((C * 64, 1024), dtype=np.float32)
    for wi, (dy, dx) in enumerate([(0, 0), (0, 1), (1, 0), (1, 1)]):
        m = np.arange(O * 9)
        o, pi, pj = m // 9, (m // 3) % 3, m % 3
        i_ = 2 * pi + dy
        j_ = 2 * pj + dx
        di = u[:, None] - i_[None, :]
        dj = v[:, None] - j_[None, :]
        ok = (di >= 0) & (di <= 1) & (dj >= 0) & (dj <= 1)
        ix = ((o[None, :] * C + c[:, None]) * 2 + np.clip(di, 0, 1)) * 2 \
            + np.clip(dj, 0, 1)
        idx[:, 256 * wi:256 * wi + O * 9] = ix
        mask[:, 256 * wi:256 * wi + O * 9] = ok
    return idx, mask


_IDX1, _MASK1 = _fold_idx_conv1_pooled()
_IDX2, _MASK2 = _fold_idx_conv(32, 16, 3, 3)
_IDX3, _MASK3 = _fold_idx_conv(64, 32, 2, 2)


def _enc_kernel(x_hbm, w0_ref, b0_ref, a1_ref, c1_ref, a2_ref, c2_ref,
                a3_ref, b3_ref, o_hbm, *, block_b, nb):
    def body(x_blk, o_blk):
        x = x_blk[...].astype(jnp.bfloat16)
        h0 = jnp.dot(x, w0_ref[...], preferred_element_type=jnp.float32) \
            + b0_ref[...]
        t = jnp.dot(h0.astype(jnp.bfloat16), a1_ref[...],
                    preferred_element_type=jnp.float32)
        m = jnp.maximum(jnp.maximum(t[:, 0:144], t[:, 256:400]),
                        jnp.maximum(t[:, 512:656], t[:, 768:912]))
        p = jnp.maximum(m + c1_ref[...], 0.0)
        h2 = jnp.maximum(
            jnp.dot(p.astype(jnp.bfloat16), a2_ref[...],
                    preferred_element_type=jnp.float32) + c2_ref[...], 0.0)
        o_blk[...] = jnp.maximum(
            jnp.dot(h2.astype(jnp.bfloat16), a3_ref[...],
                    preferred_element_type=jnp.float32) + b3_ref[...], 0.0)

    pltpu.emit_pipeline(
        body,
        grid=(nb,),
        in_specs=[pl.BlockSpec((block_b, x_hbm.shape[1]),
                               lambda i: (i, 0))],
        out_specs=[pl.BlockSpec((block_b, 64), lambda i: (i, 0))],
    )(x_hbm, o_hbm)


@functools.partial(jax.jit, static_argnames=("block_b", "interpret"))
def _encode(x, W0, b0, w1, b1, w2, b2, w3, b3, block_b=2048,
            interpret=False):
    B, D = x.shape
    W0t = W0.T.astype(jnp.bfloat16)
    a1 = (w1.ravel()[_IDX1] * _MASK1).astype(jnp.bfloat16)
    a2 = (w2.ravel()[_IDX2] * _MASK2).astype(jnp.bfloat16)
    a3 = (w3.ravel()[_IDX3] * _MASK3).astype(jnp.bfloat16)
    c1 = jnp.repeat(b1, 9).reshape(1, 144)
    c2 = jnp.repeat(b2, 4).reshape(1, 128)

    nb = B // block_b
    full = lambda *s: pl.BlockSpec(s, lambda: (0,) * len(s))
    out = pl.pallas_call(
        functools.partial(_enc_kernel, block_b=block_b, nb=nb),
        in_specs=[
            pl.BlockSpec(memory_space=pltpu.MemorySpace.HBM),
            full(D, 192),
            full(1, 192),
            full(192, 1024),
            full(1, 144),
            full(144, 128),
            full(1, 128),
            full(128, 64),
            full(1, 64),
        ],
        out_specs=pl.BlockSpec(memory_space=pltpu.MemorySpace.HBM),
        out_shape=jax.ShapeDtypeStruct((B, 64), jnp.float32),
        interpret=interpret,
    )(x, W0t, b0.reshape(1, -1), a1, c1, a2, c2, a3, b3.reshape(1, -1))
    return out.reshape(B, 64, 1, 1)


def kernel(x, W0, b0, w1, b1, w2, b2, w3, b3):
    return _encode(x, W0, b0, w1, b1, w2, b2, w3, b3)
